# Initial kernel scaffold; baseline (speedup 1.0000x reference)
import functools

import jax
import jax.numpy as jnp
from jax import lax
from jax.experimental import pallas as pl
from jax.experimental.pallas import tpu as pltpu
from jax.experimental.pallas import tpu_sc as plsc

N, E, D = 10000, 320000, 128
NC, NS = 2, 16
NW = NC * NS
EPW = E // NW
CHUNK = 128
NFULL = EPW // CHUNK
TAIL = EPW - NFULL * CHUNK
RPS = N // NS


def _sc_segment_partials(h, src, dst, zeros):
    mesh = plsc.VectorSubcoreMesh(core_axis_name="c", subcore_axis_name="s")

    @functools.partial(
        pl.kernel,
        out_type=jax.ShapeDtypeStruct((NC, N, D), jnp.float32),
        mesh=mesh,
        scratch_types=[
            pltpu.VMEM((CHUNK,), jnp.int32),
            pltpu.VMEM((CHUNK,), jnp.int32),
            pltpu.VMEM((CHUNK, D), jnp.float32),
            pltpu.VMEM((TAIL,), jnp.int32),
            pltpu.VMEM((TAIL,), jnp.int32),
            pltpu.VMEM((TAIL, D), jnp.float32),
            pltpu.VMEM_SHARED((N, D), jnp.float32),
        ],
    )
    def k(h_hbm, src_hbm, dst_hbm, zero_hbm, out_hbm,
          src_v, dst_v, rows_v, tsrc_v, tdst_v, trows_v, agg_sh):
        c = lax.axis_index("c")
        s = lax.axis_index("s")
        wid = c * NS + s
        base = wid * EPW

        pltpu.sync_copy(zero_hbm.at[pl.ds(s * RPS, RPS)],
                        agg_sh.at[pl.ds(s * RPS, RPS)])
        plsc.subcore_barrier()

        @pl.loop(0, NFULL)
        def _(i):
            off = base + i * CHUNK
            pltpu.sync_copy(src_hbm.at[pl.ds(off, CHUNK)], src_v)
            pltpu.sync_copy(dst_hbm.at[pl.ds(off, CHUNK)], dst_v)
            pltpu.sync_copy(h_hbm.at[src_v], rows_v)
            pltpu.sync_copy(rows_v, agg_sh.at[dst_v], add=True)

        toff = base + NFULL * CHUNK
        pltpu.sync_copy(src_hbm.at[pl.ds(toff, TAIL)], tsrc_v)
        pltpu.sync_copy(dst_hbm.at[pl.ds(toff, TAIL)], tdst_v)
        pltpu.sync_copy(h_hbm.at[tsrc_v], trows_v)
        pltpu.sync_copy(trows_v, agg_sh.at[tdst_v], add=True)

        plsc.subcore_barrier()
        pltpu.sync_copy(agg_sh.at[pl.ds(s * RPS, RPS)],
                        out_hbm.at[c].at[pl.ds(s * RPS, RPS)])

    return k(h, src, dst, zeros)


_BLK = 512


def _lin1(x, Wl, bl):
    def body(x_ref, w_ref, b_ref, o_ref):
        o_ref[...] = jnp.dot(x_ref[...], w_ref[...],
                             preferred_element_type=jnp.float32) + b_ref[...]

    return pl.pallas_call(
        body,
        grid=(pl.cdiv(N, _BLK),),
        in_specs=[
            pl.BlockSpec((_BLK, D), lambda i: (i, 0)),
            pl.BlockSpec((D, D), lambda i: (0, 0)),
            pl.BlockSpec((1, D), lambda i: (0, 0)),
        ],
        out_specs=pl.BlockSpec((_BLK, D), lambda i: (i, 0)),
        out_shape=jax.ShapeDtypeStruct((N, D), jnp.float32),
    )(x, Wl, bl.reshape(1, D))


def _mlp(h, parts, W1, b1, W2, b2, W3, b3):
    def body(h_ref, p_ref, w1_ref, b1_ref, w2_ref, b2_ref, w3_ref, b3_ref,
             o_ref):
        z = h_ref[...] + p_ref[0] + p_ref[1]
        z = jnp.maximum(
            jnp.dot(z, w1_ref[...], preferred_element_type=jnp.float32)
            + b1_ref[...], 0.0)
        z = jnp.maximum(
            jnp.dot(z, w2_ref[...], preferred_element_type=jnp.float32)
            + b2_ref[...], 0.0)
        z = jnp.maximum(
            jnp.dot(z, w3_ref[...], preferred_element_type=jnp.float32)
            + b3_ref[...], 0.0)
        o_ref[...] = z

    return pl.pallas_call(
        body,
        grid=(pl.cdiv(N, _BLK),),
        in_specs=[
            pl.BlockSpec((_BLK, D), lambda i: (i, 0)),
            pl.BlockSpec((NC, _BLK, D), lambda i: (0, i, 0)),
            pl.BlockSpec((D, D), lambda i: (0, 0)),
            pl.BlockSpec((1, D), lambda i: (0, 0)),
            pl.BlockSpec((D, D), lambda i: (0, 0)),
            pl.BlockSpec((1, D), lambda i: (0, 0)),
            pl.BlockSpec((D, D), lambda i: (0, 0)),
            pl.BlockSpec((1, D), lambda i: (0, 0)),
        ],
        out_specs=pl.BlockSpec((_BLK, D), lambda i: (i, 0)),
        out_shape=jax.ShapeDtypeStruct((N, D), jnp.float32),
    )(h, parts, W1, b1.reshape(1, D), W2, b2.reshape(1, D), W3,
      b3.reshape(1, D))


def kernel(x, edge_index, Wl, bl, W1, b1, W2, b2, W3, b3):
    src = edge_index[0]
    dst = edge_index[1]
    zeros = jnp.zeros((N, D), jnp.float32)
    h = _lin1(x, Wl, bl)
    parts = _sc_segment_partials(h, src, dst, zeros)
    return _mlp(h, parts, W1, b1, W2, b2, W3, b3)


# speedup vs baseline: 6.2453x; 6.2453x over previous
import functools

import jax
import jax.numpy as jnp
from jax import lax
from jax.experimental import pallas as pl
from jax.experimental.pallas import tpu as pltpu
from jax.experimental.pallas import tpu_sc as plsc

N, E, D = 10000, 320000, 128
NC, NS = 2, 16
NW = NC * NS
EPW = E // NW
CHUNK = 128
NFULL = EPW // CHUNK
TAIL = EPW - NFULL * CHUNK
RPS = 624
RREM = N - NS * RPS


def _sc_segment_partials(h, src, dst, zeros):
    mesh = plsc.VectorSubcoreMesh(core_axis_name="c", subcore_axis_name="s")

    @functools.partial(
        pl.kernel,
        out_type=jax.ShapeDtypeStruct((NC, N, D), jnp.float32),
        mesh=mesh,
        scratch_types=[
            pltpu.VMEM((CHUNK,), jnp.int32),
            pltpu.VMEM((CHUNK,), jnp.int32),
            pltpu.VMEM((CHUNK, D), jnp.float32),
            pltpu.VMEM((TAIL,), jnp.int32),
            pltpu.VMEM((TAIL,), jnp.int32),
            pltpu.VMEM((TAIL, D), jnp.float32),
            pltpu.VMEM_SHARED((N, D), jnp.float32),
        ],
    )
    def k(h_hbm, src_hbm, dst_hbm, zero_hbm, out_hbm,
          src_v, dst_v, rows_v, tsrc_v, tdst_v, trows_v, agg_sh):
        c = lax.axis_index("c")
        s = lax.axis_index("s")
        wid = c * NS + s
        base = wid * EPW

        pltpu.sync_copy(zero_hbm.at[pl.ds(s * RPS, RPS)],
                        agg_sh.at[pl.ds(s * RPS, RPS)])

        @pl.when(s == NS - 1)
        def _():
            pltpu.sync_copy(zero_hbm.at[pl.ds(NS * RPS, RREM)],
                            agg_sh.at[pl.ds(NS * RPS, RREM)])

        plsc.subcore_barrier()

        @pl.loop(0, NFULL)
        def _(i):
            off = base + i * CHUNK
            pltpu.sync_copy(src_hbm.at[pl.ds(off, CHUNK)], src_v)
            pltpu.sync_copy(dst_hbm.at[pl.ds(off, CHUNK)], dst_v)
            pltpu.sync_copy(h_hbm.at[src_v], rows_v)
            pltpu.sync_copy(rows_v, agg_sh.at[dst_v], add=True)

        toff = base + NFULL * CHUNK
        pltpu.sync_copy(src_hbm.at[pl.ds(toff, TAIL)], tsrc_v)
        pltpu.sync_copy(dst_hbm.at[pl.ds(toff, TAIL)], tdst_v)
        pltpu.sync_copy(h_hbm.at[tsrc_v], trows_v)
        pltpu.sync_copy(trows_v, agg_sh.at[tdst_v], add=True)

        plsc.subcore_barrier()
        pltpu.sync_copy(agg_sh.at[pl.ds(s * RPS, RPS)],
                        out_hbm.at[c].at[pl.ds(s * RPS, RPS)])

        @pl.when(s == NS - 1)
        def _():
            pltpu.sync_copy(agg_sh.at[pl.ds(NS * RPS, RREM)],
                            out_hbm.at[c].at[pl.ds(NS * RPS, RREM)])

    return k(h, src, dst, zeros)


_BLK = 512


def _lin1(x, Wl, bl):
    def body(x_ref, w_ref, b_ref, o_ref):
        o_ref[...] = jnp.dot(x_ref[...], w_ref[...],
                             preferred_element_type=jnp.float32) + b_ref[...]

    return pl.pallas_call(
        body,
        grid=(pl.cdiv(N, _BLK),),
        in_specs=[
            pl.BlockSpec((_BLK, D), lambda i: (i, 0)),
            pl.BlockSpec((D, D), lambda i: (0, 0)),
            pl.BlockSpec((1, D), lambda i: (0, 0)),
        ],
        out_specs=pl.BlockSpec((_BLK, D), lambda i: (i, 0)),
        out_shape=jax.ShapeDtypeStruct((N, D), jnp.float32),
    )(x, Wl, bl.reshape(1, D))


def _mlp(h, parts, W1, b1, W2, b2, W3, b3):
    def body(h_ref, p_ref, w1_ref, b1_ref, w2_ref, b2_ref, w3_ref, b3_ref,
             o_ref):
        z = h_ref[...] + p_ref[0] + p_ref[1]
        z = jnp.maximum(
            jnp.dot(z, w1_ref[...], preferred_element_type=jnp.float32)
            + b1_ref[...], 0.0)
        z = jnp.maximum(
            jnp.dot(z, w2_ref[...], preferred_element_type=jnp.float32)
            + b2_ref[...], 0.0)
        z = jnp.maximum(
            jnp.dot(z, w3_ref[...], preferred_element_type=jnp.float32)
            + b3_ref[...], 0.0)
        o_ref[...] = z

    return pl.pallas_call(
        body,
        grid=(pl.cdiv(N, _BLK),),
        in_specs=[
            pl.BlockSpec((_BLK, D), lambda i: (i, 0)),
            pl.BlockSpec((NC, _BLK, D), lambda i: (0, i, 0)),
            pl.BlockSpec((D, D), lambda i: (0, 0)),
            pl.BlockSpec((1, D), lambda i: (0, 0)),
            pl.BlockSpec((D, D), lambda i: (0, 0)),
            pl.BlockSpec((1, D), lambda i: (0, 0)),
            pl.BlockSpec((D, D), lambda i: (0, 0)),
            pl.BlockSpec((1, D), lambda i: (0, 0)),
        ],
        out_specs=pl.BlockSpec((_BLK, D), lambda i: (i, 0)),
        out_shape=jax.ShapeDtypeStruct((N, D), jnp.float32),
    )(h, parts, W1, b1.reshape(1, D), W2, b2.reshape(1, D), W3,
      b3.reshape(1, D))


def kernel(x, edge_index, Wl, bl, W1, b1, W2, b2, W3, b3):
    src = edge_index[0]
    dst = edge_index[1]
    zeros = jnp.zeros((N, D), jnp.float32)
    h = _lin1(x, Wl, bl)
    parts = _sc_segment_partials(h, src, dst, zeros)
    return _mlp(h, parts, W1, b1, W2, b2, W3, b3)
